# baseline (speedup 1.0000x reference)
import jax
import jax.numpy as jnp
from jax import lax
from jax.experimental import pallas as pl
from jax.experimental.pallas import tpu as pltpu

_B = 128
_S = 2048
_D = 1024
_ROWS = 2 * _B


def _tc_body(pos1_ref, pos2_ref, base_ref, out_ref, sem):
    def issue(i, _):
        p1 = pos1_ref[i]
        p2 = pos2_ref[i]
        pltpu.make_async_copy(base_ref.at[i, p1], out_ref.at[2 * i], sem).start()
        pltpu.make_async_copy(base_ref.at[i, p2], out_ref.at[2 * i + 1], sem).start()
        return 0

    lax.fori_loop(0, _B, issue, 0, unroll=8)

    def drain(i, _):
        pltpu.make_async_copy(base_ref.at[i, 0], out_ref.at[2 * i], sem).wait()
        pltpu.make_async_copy(base_ref.at[i, 0], out_ref.at[2 * i + 1], sem).wait()
        return 0

    lax.fori_loop(0, _B, drain, 0, unroll=8)


@jax.jit
def kernel(base_encoding, pos1, pos2):
    out = pl.pallas_call(
        _tc_body,
        grid=(),
        in_specs=[
            pl.BlockSpec(memory_space=pltpu.SMEM),
            pl.BlockSpec(memory_space=pltpu.SMEM),
            pl.BlockSpec(memory_space=pl.ANY),
        ],
        out_specs=pl.BlockSpec(memory_space=pl.ANY),
        out_shape=jax.ShapeDtypeStruct((_ROWS, _D), jnp.float32),
        scratch_shapes=[pltpu.SemaphoreType.DMA],
    )(pos1.astype(jnp.int32), pos2.astype(jnp.int32), base_encoding)
    return out.reshape(_B, 2 * _D)


# speedup vs baseline: 5.3757x; 5.3757x over previous
import jax
import jax.numpy as jnp
from jax import lax
from jax.experimental import pallas as pl
from jax.experimental.pallas import tpu as pltpu

_B = 128
_S = 2048
_D = 1024
_ROWS = 2 * _B


def _tc_body(pos1_ref, pos2_ref, base_ref, out_ref, sem):
    def issue(i, _):
        p1 = pos1_ref[i]
        p2 = pos2_ref[i]
        pltpu.make_async_copy(base_ref.at[i, p1], out_ref.at[2 * i], sem).start()
        pltpu.make_async_copy(base_ref.at[i, p2], out_ref.at[2 * i + 1], sem).start()
        return 0

    lax.fori_loop(0, _B, issue, 0, unroll=8)

    def drain(i, _):
        pltpu.make_async_copy(base_ref.at[i, 0], out_ref.at[2 * i], sem).wait()
        pltpu.make_async_copy(base_ref.at[i, 0], out_ref.at[2 * i + 1], sem).wait()
        return 0

    lax.fori_loop(0, _B, drain, 0, unroll=8)


@jax.jit
def kernel(base_encoding, pos1, pos2):
    out = pl.pallas_call(
        _tc_body,
        grid=(),
        in_specs=[
            pl.BlockSpec(memory_space=pltpu.SMEM),
            pl.BlockSpec(memory_space=pltpu.SMEM),
            pl.BlockSpec(memory_space=pl.ANY),
        ],
        out_specs=pl.BlockSpec(memory_space=pltpu.VMEM),
        out_shape=jax.ShapeDtypeStruct((_ROWS, _D), jnp.float32),
        scratch_shapes=[pltpu.SemaphoreType.DMA],
    )(pos1.astype(jnp.int32), pos2.astype(jnp.int32), base_encoding)
    return out.reshape(_B, 2 * _D)
